# Initial kernel scaffold; baseline (speedup 1.0000x reference)
import jax
import jax.numpy as jnp
from jax.experimental import pallas as pl


def kernel(inputs, embeddings):
    raise NotImplementedError("write your pallas kernel here")


# speedup vs baseline: 3.4599x; 3.4599x over previous
import functools

import jax
import jax.numpy as jnp
from jax import lax
from jax.experimental import pallas as pl
from jax.experimental.pallas import tpu as pltpu
from jax.experimental.pallas import tpu_sc as plsc

NC = 2
NS = 16
NW = NC * NS

ROWS = 16384 * 50
D = 128
CHUNK = 128
PER_W = ROWS // NW
N_CHUNKS = PER_W // CHUNK
NBUF = 4
N_GROUPS = N_CHUNKS // NBUF

_mesh = plsc.VectorSubcoreMesh(
    core_axis_name="c", subcore_axis_name="s", num_cores=NC, num_subcores=NS
)


@functools.partial(
    pl.kernel,
    out_type=jax.ShapeDtypeStruct((ROWS, D), jnp.float32),
    mesh=_mesh,
    scratch_types=[
        pltpu.VMEM((N_CHUNKS, CHUNK), jnp.int32),
        [pltpu.VMEM((CHUNK, D), jnp.float32) for _ in range(NBUF)],
        [pltpu.SemaphoreType.DMA for _ in range(NBUF)],
    ],
)
def _sc_gather(idx_hbm, table_hbm, out_hbm, idx_v, rows, sems):
    wid = lax.axis_index("s") * NC + lax.axis_index("c")
    chunk0 = wid * N_CHUNKS
    row0 = chunk0 * CHUNK

    pltpu.sync_copy(idx_hbm.at[pl.ds(chunk0, N_CHUNKS)], idx_v)

    def fire(j, b):
        pltpu.async_copy(table_hbm.at[idx_v.at[j]], rows[b], sems[b])

    def wait(b):
        pltpu.make_async_copy(table_hbm.at[idx_v.at[0]], rows[b], sems[b]).wait()

    for b in range(NBUF):
        fire(b, b)

    def group(g, _):
        for b in range(NBUF):
            j = g * NBUF + b
            wait(b)
            pltpu.sync_copy(rows[b], out_hbm.at[pl.ds(row0 + j * CHUNK, CHUNK)])
            fire(j + NBUF, b)
        return _

    lax.fori_loop(0, N_GROUPS - 1, group, 0, unroll=False)

    for b in range(NBUF):
        j = (N_GROUPS - 1) * NBUF + b
        wait(b)
        pltpu.sync_copy(rows[b], out_hbm.at[pl.ds(row0 + j * CHUNK, CHUNK)])


def kernel(inputs, embeddings):
    idx = inputs.reshape(ROWS // CHUNK, CHUNK).astype(jnp.int32)
    out = _sc_gather(idx, embeddings)
    return out.reshape(inputs.shape[0], inputs.shape[1], D)
